# Initial kernel scaffold; baseline (speedup 1.0000x reference)
import jax
import jax.numpy as jnp
from jax.experimental import pallas as pl


def kernel(x, emb_table, W_epoch, W_cfg, b_cfg):
    raise NotImplementedError("write your pallas kernel here")


# speedup vs baseline: 3.1982x; 3.1982x over previous
import math

import jax
import jax.numpy as jnp
from jax.experimental import pallas as pl
from jax.experimental.pallas import tpu as pltpu

IN_DIM = 34
OUT_DIM = 256
N_EMB_PAD = 1024
TB = 1024


def _body(x_ref, tab_ref, w_ref, b_ref, out_ref):
    xb = x_ref[...]
    ids = xb[:, 0].astype(jnp.int32)
    oh = (ids[:, None] == jax.lax.broadcasted_iota(
        jnp.int32, (TB, N_EMB_PAD), 1)).astype(jnp.bfloat16)
    rows = jax.lax.dot_general(
        oh, tab_ref[...], (((1,), (0,)), ((), ())),
        preferred_element_type=jnp.float32,
    )
    dense = jax.lax.dot_general(
        xb, w_ref[...], (((1,), (0,)), ((), ())),
        preferred_element_type=jnp.float32,
    )
    out_ref[...] = rows + dense + b_ref[...]


def kernel(x, emb_table, W_epoch, W_cfg, b_cfg):
    S, B, _ = x.shape
    n_tok = S * B
    xf = x.reshape(n_tok, IN_DIM)

    std = math.sqrt(1.0 / 12.0)
    w_e = W_epoch[:, 0]
    w_comb = jnp.concatenate(
        [jnp.zeros((1, OUT_DIM), jnp.float32), (w_e / std)[None, :], W_cfg.T],
        axis=0,
    )
    bias = (b_cfg - (0.5 / std) * w_e)[None, :]
    tab = jnp.pad(emb_table, ((0, N_EMB_PAD - emb_table.shape[0]), (0, 0))
                  ).astype(jnp.bfloat16)

    grid = (n_tok // TB,)
    out = pl.pallas_call(
        _body,
        grid=grid,
        in_specs=[
            pl.BlockSpec((TB, IN_DIM), lambda i: (i, 0)),
            pl.BlockSpec((N_EMB_PAD, OUT_DIM), lambda i: (0, 0)),
            pl.BlockSpec((IN_DIM, OUT_DIM), lambda i: (0, 0)),
            pl.BlockSpec((1, OUT_DIM), lambda i: (0, 0)),
        ],
        out_specs=pl.BlockSpec((TB, OUT_DIM), lambda i: (i, 0)),
        out_shape=jax.ShapeDtypeStruct((n_tok, OUT_DIM), jnp.float32),
    )(xf, tab, w_comb, bias)
    return out.reshape(S, B, OUT_DIM)
